# baseline (speedup 1.0000x reference)
import functools

import jax
import jax.numpy as jnp
from jax import lax
from jax.experimental import pallas as pl
from jax.experimental.pallas import tpu as pltpu
from jax.experimental.pallas import tpu_sc as plsc

B = 4096
N_FIELDS = 26
FIELD_DIM = 10000
TOTAL = N_FIELDS * FIELD_DIM
EMB = 64
MLP_IN = N_FIELDS * EMB
_OFFSETS = jnp.arange(N_FIELDS, dtype=jnp.int32) * FIELD_DIM

NW = 32
NSPLIT = 2
BH = B // NSPLIT
TOT_IDX = BH * N_FIELDS
PER_W = TOT_IDX // NW
CHUNK = 128
NCH = PER_W // CHUNK


def _sc_body(emb_hbm, lin_hbm, idx_hbm, emb_out, lin_out,
             idx_v, rows0, rows1, lin_all, sem_g0, sem_g1, sem_w0, sem_w1,
             sem_l):
    wid = lax.axis_index("s") * 2 + lax.axis_index("c")
    pltpu.sync_copy(idx_hbm.at[wid], idx_v)
    base = wid * PER_W
    rows = (rows0, rows1)
    semg = (sem_g0, sem_g1)
    semw = (sem_w0, sem_w1)

    for j in range(NCH):
        pltpu.async_copy(lin_hbm.at[idx_v.at[j]],
                         lin_all.at[pl.ds(j * CHUNK, CHUNK)], sem_l)

    pltpu.async_copy(emb_hbm.at[idx_v.at[0]], rows0, sem_g0)
    if NCH > 1:
        pltpu.async_copy(emb_hbm.at[idx_v.at[1]], rows1, sem_g1)
    for j in range(NCH):
        b = j % 2
        pltpu.make_async_copy(emb_hbm.at[idx_v.at[j]], rows[b],
                              semg[b]).wait()
        off = pl.multiple_of(base + j * CHUNK, CHUNK)
        pltpu.async_copy(rows[b], emb_out.at[pl.ds(off, CHUNK)], semw[b])
        if j + 2 < NCH:
            pltpu.make_async_copy(rows[b], emb_out.at[pl.ds(off, CHUNK)],
                                  semw[b]).wait()
            pltpu.async_copy(emb_hbm.at[idx_v.at[j + 2]], rows[b], semg[b])
    for j in (NCH - 2, NCH - 1):
        if j >= 0:
            b = j % 2
            off = pl.multiple_of(base + j * CHUNK, CHUNK)
            pltpu.make_async_copy(rows[b], emb_out.at[pl.ds(off, CHUNK)],
                                  semw[b]).wait()

    for j in range(NCH):
        pltpu.make_async_copy(lin_hbm.at[idx_v.at[j]],
                              lin_all.at[pl.ds(j * CHUNK, CHUNK)],
                              sem_l).wait()
    pltpu.sync_copy(lin_all, lin_out.at[pl.ds(base, PER_W)])


@functools.lru_cache(maxsize=None)
def _get_sc_gather():
    return pl.kernel(
        _sc_body,
        out_type=[
            jax.ShapeDtypeStruct((TOT_IDX, EMB), jnp.float32),
            jax.ShapeDtypeStruct((TOT_IDX,), jnp.float32),
        ],
        mesh=plsc.VectorSubcoreMesh(core_axis_name="c", subcore_axis_name="s"),
        compiler_params=pltpu.CompilerParams(use_tc_tiling_on_sc=False),
        scratch_types=[
            pltpu.VMEM((NCH, CHUNK), jnp.int32),
            pltpu.VMEM((CHUNK, EMB), jnp.float32),
            pltpu.VMEM((CHUNK, EMB), jnp.float32),
            pltpu.VMEM((PER_W,), jnp.float32),
            pltpu.SemaphoreType.DMA,
            pltpu.SemaphoreType.DMA,
            pltpu.SemaphoreType.DMA,
            pltpu.SemaphoreType.DMA,
            pltpu.SemaphoreType.DMA,
        ],
    )


BB = 512


def _tc_body(emb_ref, lin_ref, s_ref, w1_ref, b1_ref, w2_ref, b2_ref,
             w3_ref, b3_ref, bias_ref, out_ref):
    x = emb_ref[...]
    h = jnp.dot(x, w1_ref[...], preferred_element_type=jnp.float32)
    h = jnp.maximum(h + b1_ref[...], 0.0)
    h = jnp.dot(h, w2_ref[...], preferred_element_type=jnp.float32)
    h = jnp.maximum(h + b2_ref[...], 0.0)
    mlp = jnp.dot(h, w3_ref[...], preferred_element_type=jnp.float32)
    mlp = mlp + b3_ref[...]
    sum_e = jnp.dot(x, s_ref[...], preferred_element_type=jnp.float32)
    sq = jnp.sum(sum_e * sum_e, axis=1, keepdims=True)
    q = jnp.sum(x * x, axis=1, keepdims=True)
    fm = 0.5 * (sq - q)
    lin = jnp.sum(lin_ref[...], axis=1, keepdims=True) + bias_ref[...]
    out_ref[...] = jax.nn.sigmoid(lin + fm + mlp)


def _tc_call(emb2, lin2, smat, W1, b1, W2, b2, W3, b3, bias):
    return pl.pallas_call(
        _tc_body,
        grid=(BH // BB,),
        in_specs=[
            pl.BlockSpec((BB, MLP_IN), lambda i: (i, 0)),
            pl.BlockSpec((BB, N_FIELDS), lambda i: (i, 0)),
            pl.BlockSpec((MLP_IN, EMB), lambda i: (0, 0)),
            pl.BlockSpec((MLP_IN, 32), lambda i: (0, 0)),
            pl.BlockSpec((1, 32), lambda i: (0, 0)),
            pl.BlockSpec((32, 32), lambda i: (0, 0)),
            pl.BlockSpec((1, 32), lambda i: (0, 0)),
            pl.BlockSpec((32, 1), lambda i: (0, 0)),
            pl.BlockSpec((1, 1), lambda i: (0, 0)),
            pl.BlockSpec((1, 1), lambda i: (0, 0)),
        ],
        out_specs=pl.BlockSpec((BB, 1), lambda i: (i, 0)),
        out_shape=jax.ShapeDtypeStruct((BH, 1), jnp.float32),
    )(emb2, lin2, smat, W1, b1, W2, b2, W3, b3, bias)


def kernel(x, emb_table, lin_table, bias, W1, b1, W2, b2, W3, b3):
    idx = (x + _OFFSETS[None, :]).reshape(NSPLIT, NW, NCH, CHUNK)
    lin_flat = lin_table.reshape(TOTAL)
    sc = _get_sc_gather()
    smat = jnp.tile(jnp.eye(EMB, dtype=jnp.float32), (N_FIELDS, 1))
    b1r = b1.reshape(1, 32)
    b2r = b2.reshape(1, 32)
    b3r = b3.reshape(1, 1)
    biasr = bias.reshape(1, 1)
    outs = []
    for s in range(NSPLIT):
        emb_g, lin_g = sc(emb_table, lin_flat, idx[s])
        emb2 = emb_g.reshape(BH, MLP_IN)
        lin2 = lin_g.reshape(BH, N_FIELDS)
        outs.append(_tc_call(emb2, lin2, smat, W1, b1r, W2, b2r, W3, b3r,
                             biasr))
    return jnp.concatenate(outs, axis=0)[:, 0]


# speedup vs baseline: 1.0048x; 1.0048x over previous
import functools

import jax
import jax.numpy as jnp
from jax import lax
from jax.experimental import pallas as pl
from jax.experimental.pallas import tpu as pltpu
from jax.experimental.pallas import tpu_sc as plsc

B = 4096
N_FIELDS = 26
FIELD_DIM = 10000
TOTAL = N_FIELDS * FIELD_DIM
EMB = 64
MLP_IN = N_FIELDS * EMB
_OFFSETS = jnp.arange(N_FIELDS, dtype=jnp.int32) * FIELD_DIM

NW = 32
NSPLIT = 2
BH = B // NSPLIT
TOT_IDX = BH * N_FIELDS
PER_W = TOT_IDX // NW
CHUNK = 128
NCH = PER_W // CHUNK


def _sc_body(emb_hbm, lin_hbm, idx_hbm, emb_out, lin_out,
             idx_v, rows0, rows1, lin_all, sem_g0, sem_g1, sem_w0, sem_w1,
             sem_l):
    wid = lax.axis_index("s") * 2 + lax.axis_index("c")
    pltpu.sync_copy(idx_hbm.at[wid], idx_v)
    base = wid * PER_W
    rows = (rows0, rows1)
    semg = (sem_g0, sem_g1)
    semw = (sem_w0, sem_w1)

    for j in range(NCH):
        pltpu.async_copy(lin_hbm.at[idx_v.at[j]],
                         lin_all.at[pl.ds(j * CHUNK, CHUNK)], sem_l)

    pltpu.async_copy(emb_hbm.at[idx_v.at[0]], rows0, sem_g0)
    if NCH > 1:
        pltpu.async_copy(emb_hbm.at[idx_v.at[1]], rows1, sem_g1)
    for j in range(NCH):
        b = j % 2
        pltpu.make_async_copy(emb_hbm.at[idx_v.at[j]], rows[b],
                              semg[b]).wait()
        off = pl.multiple_of(base + j * CHUNK, CHUNK)
        pltpu.async_copy(rows[b], emb_out.at[pl.ds(off, CHUNK)], semw[b])
        if j + 2 < NCH:
            pltpu.make_async_copy(rows[b], emb_out.at[pl.ds(off, CHUNK)],
                                  semw[b]).wait()
            pltpu.async_copy(emb_hbm.at[idx_v.at[j + 2]], rows[b], semg[b])
    for j in (NCH - 2, NCH - 1):
        if j >= 0:
            b = j % 2
            off = pl.multiple_of(base + j * CHUNK, CHUNK)
            pltpu.make_async_copy(rows[b], emb_out.at[pl.ds(off, CHUNK)],
                                  semw[b]).wait()

    for j in range(NCH):
        pltpu.make_async_copy(lin_hbm.at[idx_v.at[j]],
                              lin_all.at[pl.ds(j * CHUNK, CHUNK)],
                              sem_l).wait()
    pltpu.sync_copy(lin_all, lin_out.at[pl.ds(base, PER_W)])


@functools.lru_cache(maxsize=None)
def _get_sc_gather():
    return pl.kernel(
        _sc_body,
        out_type=[
            jax.ShapeDtypeStruct((TOT_IDX, EMB), jnp.float32),
            jax.ShapeDtypeStruct((TOT_IDX,), jnp.float32),
        ],
        mesh=plsc.VectorSubcoreMesh(core_axis_name="c", subcore_axis_name="s"),
        compiler_params=pltpu.CompilerParams(use_tc_tiling_on_sc=False),
        scratch_types=[
            pltpu.VMEM((NCH, CHUNK), jnp.int32),
            pltpu.VMEM((CHUNK, EMB), jnp.float32),
            pltpu.VMEM((CHUNK, EMB), jnp.float32),
            pltpu.VMEM((PER_W,), jnp.float32),
            pltpu.SemaphoreType.DMA,
            pltpu.SemaphoreType.DMA,
            pltpu.SemaphoreType.DMA,
            pltpu.SemaphoreType.DMA,
            pltpu.SemaphoreType.DMA,
        ],
    )


BB = 512


def _tc_body(emb_ref, lin_ref, s_ref, w1_ref, b1_ref, w2_ref, b2_ref,
             w3_ref, b3_ref, bias_ref, out_ref):
    x = emb_ref[...]
    h = jnp.dot(x, w1_ref[...], preferred_element_type=jnp.float32)
    h = jnp.maximum(h + b1_ref[...], 0.0)
    h = jnp.dot(h, w2_ref[...], preferred_element_type=jnp.float32)
    h = jnp.maximum(h + b2_ref[...], 0.0)
    mlp = jnp.dot(h, w3_ref[...], preferred_element_type=jnp.float32)
    mlp = mlp + b3_ref[...]
    sum_e = jnp.dot(x, s_ref[...], preferred_element_type=jnp.float32)
    sq = jnp.sum(sum_e * sum_e, axis=1, keepdims=True)
    q = jnp.sum(x * x, axis=1, keepdims=True)
    fm = 0.5 * (sq - q)
    lin = jnp.sum(lin_ref[...], axis=1, keepdims=True) + bias_ref[...]
    out_ref[...] = jax.nn.sigmoid(lin + fm + mlp)


def _tc_call(emb2, lin2, smat, W1, b1, W2, b2, W3, b3, bias):
    return pl.pallas_call(
        _tc_body,
        grid=(BH // BB,),
        in_specs=[
            pl.BlockSpec((BB, MLP_IN), lambda i: (i, 0)),
            pl.BlockSpec((BB, N_FIELDS), lambda i: (i, 0)),
            pl.BlockSpec((MLP_IN, EMB), lambda i: (0, 0)),
            pl.BlockSpec((MLP_IN, 32), lambda i: (0, 0)),
            pl.BlockSpec((1, 32), lambda i: (0, 0)),
            pl.BlockSpec((32, 32), lambda i: (0, 0)),
            pl.BlockSpec((1, 32), lambda i: (0, 0)),
            pl.BlockSpec((32, 1), lambda i: (0, 0)),
            pl.BlockSpec((1, 1), lambda i: (0, 0)),
            pl.BlockSpec((1, 1), lambda i: (0, 0)),
        ],
        out_specs=pl.BlockSpec((BB, 1), lambda i: (i, 0)),
        out_shape=jax.ShapeDtypeStruct((BH, 1), jnp.float32),
    )(emb2, lin2, smat, W1, b1, W2, b2, W3, b3, bias)


def kernel(x, emb_table, lin_table, bias, W1, b1, W2, b2, W3, b3):
    idx = (x + _OFFSETS[None, :]).reshape(NSPLIT, NW, NCH, CHUNK)
    lin_flat = lin_table.reshape(TOTAL)
    emb_lin = jax.lax.optimization_barrier(
        emb_table.reshape(TOTAL * EMB)).reshape(TOTAL, EMB)
    sc = _get_sc_gather()
    smat = jnp.tile(jnp.eye(EMB, dtype=jnp.float32), (N_FIELDS, 1))
    b1r = b1.reshape(1, 32)
    b2r = b2.reshape(1, 32)
    b3r = b3.reshape(1, 1)
    biasr = bias.reshape(1, 1)
    outs = []
    for s in range(NSPLIT):
        emb_g, lin_g = sc(emb_lin, lin_flat, idx[s])
        emb2 = emb_g.reshape(BH, MLP_IN)
        lin2 = lin_g.reshape(BH, N_FIELDS)
        outs.append(_tc_call(emb2, lin2, smat, W1, b1r, W2, b2r, W3, b3r,
                             biasr))
    return jnp.concatenate(outs, axis=0)[:, 0]
